# baseline (speedup 1.0000x reference)
import jax
import jax.numpy as jnp
from jax.experimental import pallas as pl
from jax.experimental.pallas import tpu as pltpu

S = 2048
H = 16
D = 1024
P = 256
K = 102
BLK = 256
NBLK = S // BLK
LN_EPS = 1e-5
HI_BITS = 0x41800000


def _select_body(att_ref, len_ref, c_ref, acc_ref, prev_ref, lo_ref, hi_ref):
    b = pl.program_id(0)
    h = pl.program_id(1)

    @pl.when(jnp.logical_and(b == 0, h == 0))
    def _():
        c_ref[...] = jnp.zeros_like(c_ref)

    blk = (att_ref[0, 0, :, :] + att_ref[0, 1, :, :]
           + att_ref[0, 2, :, :] + att_ref[0, 3, :, :]
           + att_ref[0, 4, :, :] + att_ref[0, 5, :, :]
           + att_ref[0, 6, :, :] + att_ref[0, 7, :, :])

    @pl.when(h == 0)
    def _():
        acc_ref[...] = blk

    @pl.when(h > 0)
    def _():
        acc_ref[...] += blk

    @pl.when(b >= 1)
    def _():
        bits = jax.lax.bitcast_convert_type(prev_ref[...], jnp.int32)
        lo = lo_ref[...]
        hi = hi_ref[...]
        for _ in range(16):
            mid = lo + jax.lax.shift_right_logical(hi - lo, 1)
            neg_lt = jnp.sum(jax.lax.shift_right_arithmetic(bits - mid, 31),
                             axis=1, keepdims=True)
            ge = (S + neg_lt) >= K
            lo = jnp.where(ge, mid, lo)
            hi = jnp.where(ge, hi, mid)
        lo_ref[...] = lo
        hi_ref[...] = hi

    def _finalize(bits, t, row_base):
        gt = bits > t
        n_gt = jnp.sum(gt.astype(jnp.int32), axis=1, keepdims=True)
        eq = bits == t
        need = K - n_gt
        col = jax.lax.broadcasted_iota(jnp.int32, (BLK, S), 1)
        lo2 = jnp.zeros((BLK, 1), jnp.int32)
        hi2 = jnp.full((BLK, 1), S, jnp.int32)
        for _ in range(11):
            mid2 = lo2 + jax.lax.shift_right_logical(hi2 - lo2, 1)
            f = jnp.sum(jnp.logical_and(eq, col < mid2).astype(jnp.int32),
                        axis=1, keepdims=True)
            ge2 = f >= need
            hi2 = jnp.where(ge2, mid2, hi2)
            lo2 = jnp.where(ge2, lo2, mid2)
        tie_sel = eq & (col < hi2)
        topk = gt | tie_sel
        t_pos = t > 0
        pos_case = jnp.logical_and(jnp.logical_not(t_pos), n_gt > 0)
        zero_case = jnp.logical_and(jnp.logical_not(t_pos), n_gt == 0)
        sel = (jnp.logical_and(t_pos, topk)
               | jnp.logical_and(pos_case, bits > 0)
               | jnp.logical_and(zero_case, tie_sel))
        cnt_valid = jnp.where(pos_case, n_gt, K).astype(jnp.float32)
        row = (row_base
               + jax.lax.broadcasted_iota(jnp.int32, (BLK, 1), 0))
        row_valid = (row < len_ref[0, 0]).astype(jnp.float32)
        r = row_valid / cnt_valid
        contrib = jnp.sum(sel.astype(jnp.float32) * r, axis=0,
                          keepdims=True)
        c_ref[...] += contrib

    @pl.when(jnp.logical_and(h == H // 8 - 1, b >= 1))
    def _():
        bits = jax.lax.bitcast_convert_type(prev_ref[...], jnp.int32)
        _finalize(bits, lo_ref[...], (b - 1) * BLK)

    @pl.when(jnp.logical_and(h == H // 8 - 1, b < NBLK - 1))
    def _():
        prev_ref[...] = acc_ref[...]
        lo_ref[...] = jnp.zeros((BLK, 1), jnp.int32)
        hi_ref[...] = jnp.full((BLK, 1), HI_BITS, jnp.int32)

    @pl.when(jnp.logical_and(h == H // 8 - 1, b == NBLK - 1))
    def _():
        bits = jax.lax.bitcast_convert_type(acc_ref[...], jnp.int32)
        lo = jnp.zeros((BLK, 1), jnp.int32)
        hi = jnp.full((BLK, 1), HI_BITS, jnp.int32)
        for _ in range(31):
            mid = lo + jax.lax.shift_right_logical(hi - lo, 1)
            neg_lt = jnp.sum(jax.lax.shift_right_arithmetic(bits - mid, 31),
                             axis=1, keepdims=True)
            ge = (S + neg_lt) >= K
            lo = jnp.where(ge, mid, lo)
            hi = jnp.where(ge, hi, mid)
        _finalize(bits, lo, (NBLK - 1) * BLK)


def _tail_body(hid_ref, c_ref, len_ref, eta_ref, tW_ref, tb_ref, fW_ref,
               fb_ref, g_ref, bln_ref, oW_ref, ob_ref, y_ref):
    eta = eta_ref[0, 0]
    len_i = len_ref[0, 0]
    len_f = jnp.maximum(len_i, 1).astype(jnp.float32)
    col = jax.lax.broadcasted_iota(jnp.int32, (1, S), 1)
    valid = (col < len_i).astype(jnp.float32)
    u = (eta * valid + (1.0 - eta) * c_ref[...]) / len_f
    su = jnp.sum(u)
    s_vec = jnp.dot(u, hid_ref[0, :, :],
                    preferred_element_type=jnp.float32)
    gm = jnp.dot(s_vec, tW_ref[...],
                 preferred_element_type=jnp.float32) + su * tb_ref[...]
    t1 = jnp.tanh(gm)
    o = jnp.dot(t1, fW_ref[...],
                preferred_element_type=jnp.float32) + fb_ref[...]
    m = jnp.mean(o, axis=-1, keepdims=True)
    v = jnp.mean((o - m) ** 2, axis=-1, keepdims=True)
    ln = (o - m) * jax.lax.rsqrt(v + LN_EPS) * g_ref[...] + bln_ref[...]
    y_ref[...] = jnp.tanh(
        jnp.dot(ln, oW_ref[...], preferred_element_type=jnp.float32)
        + ob_ref[...])


def _module(hidden, attention, len_2d, trans_W, trans_b, fc_W, fc_b, ln_g,
            ln_b, eta_2d, out_W, out_b):
    c = pl.pallas_call(
        _select_body,
        grid=(NBLK, H // 8),
        in_specs=[
            pl.BlockSpec((1, 8, BLK, S), lambda b, h: (0, h, b, 0)),
            pl.BlockSpec(memory_space=pltpu.SMEM),
        ],
        out_specs=pl.BlockSpec((1, S), lambda b, h: (0, 0)),
        out_shape=jax.ShapeDtypeStruct((1, S), jnp.float32),
        scratch_shapes=[pltpu.VMEM((BLK, S), jnp.float32),
                        pltpu.VMEM((BLK, S), jnp.float32),
                        pltpu.VMEM((BLK, 1), jnp.int32),
                        pltpu.VMEM((BLK, 1), jnp.int32)],
    )(attention, len_2d)

    y = pl.pallas_call(
        _tail_body,
        in_specs=[
            pl.BlockSpec((1, S, D), lambda: (0, 0, 0)),
            pl.BlockSpec((1, S), lambda: (0, 0)),
            pl.BlockSpec(memory_space=pltpu.SMEM),
            pl.BlockSpec(memory_space=pltpu.SMEM),
            pl.BlockSpec((D, P), lambda: (0, 0)),
            pl.BlockSpec((1, P), lambda: (0, 0)),
            pl.BlockSpec((P, P), lambda: (0, 0)),
            pl.BlockSpec((1, P), lambda: (0, 0)),
            pl.BlockSpec((1, P), lambda: (0, 0)),
            pl.BlockSpec((1, P), lambda: (0, 0)),
            pl.BlockSpec((P, P), lambda: (0, 0)),
            pl.BlockSpec((1, P), lambda: (0, 0)),
        ],
        out_specs=pl.BlockSpec((1, P), lambda: (0, 0)),
        out_shape=jax.ShapeDtypeStruct((1, P), jnp.float32),
    )(hidden, c, len_2d, eta_2d, trans_W, trans_b.reshape(1, P), fc_W,
      fc_b.reshape(1, P), ln_g.reshape(1, P), ln_b.reshape(1, P), out_W,
      out_b.reshape(1, P))
    return y


def kernel(word_embed, word_attention, semantic_embed, semantic_attention,
           lengths, w_trans_W, w_trans_b, w_fc_W, w_fc_b, w_ln_g, w_ln_b,
           w_eta, s_trans_W, s_trans_b, s_fc_W, s_fc_b, s_ln_g, s_ln_b,
           s_eta, word_fc_W, word_fc_b, sem_fc_W, sem_fc_b):
    len_2d = lengths.reshape(1, 1)
    word_out = _module(word_embed, word_attention, len_2d, w_trans_W,
                       w_trans_b, w_fc_W, w_fc_b, w_ln_g, w_ln_b,
                       w_eta.reshape(1, 1), word_fc_W, word_fc_b)
    sem_out = _module(semantic_embed, semantic_attention, len_2d, s_trans_W,
                      s_trans_b, s_fc_W, s_fc_b, s_ln_g, s_ln_b,
                      s_eta.reshape(1, 1), sem_fc_W, sem_fc_b)
    return (word_out, sem_out)


# speedup vs baseline: 1.0312x; 1.0312x over previous
import jax
import jax.numpy as jnp
from jax.experimental import pallas as pl
from jax.experimental.pallas import tpu as pltpu

S = 2048
H = 16
D = 1024
P = 256
K = 102
BLK = 256
NBLK = S // BLK
LN_EPS = 1e-5
HI_BITS = 0x41800000


def _select_body(att_ref, len_ref, c_ref, acc_ref, prev_ref, lo_ref, hi_ref):
    b = pl.program_id(0)
    h = pl.program_id(1)

    @pl.when(jnp.logical_and(b == 0, h == 0))
    def _():
        c_ref[...] = jnp.zeros_like(c_ref)

    @pl.when(b < NBLK)
    def _():
        blk = (att_ref[0, 0, :, :] + att_ref[0, 1, :, :]
               + att_ref[0, 2, :, :] + att_ref[0, 3, :, :]
               + att_ref[0, 4, :, :] + att_ref[0, 5, :, :]
               + att_ref[0, 6, :, :] + att_ref[0, 7, :, :])

        @pl.when(h == 0)
        def _():
            acc_ref[...] = blk

        @pl.when(h > 0)
        def _():
            acc_ref[...] += blk

    @pl.when(b >= 1)
    def _():
        bits = jax.lax.bitcast_convert_type(prev_ref[...], jnp.int32)
        lo = lo_ref[...]
        hi = hi_ref[...]
        for _ in range(16):
            mid = lo + jax.lax.shift_right_logical(hi - lo, 1)
            neg_lt = jnp.sum(jax.lax.shift_right_arithmetic(bits - mid, 31),
                             axis=1, keepdims=True)
            ge = (S + neg_lt) >= K
            lo = jnp.where(ge, mid, lo)
            hi = jnp.where(ge, hi, mid)
        lo_ref[...] = lo
        hi_ref[...] = hi

    def _finalize(bits, t, row_base):
        gt = bits > t
        n_gt = jnp.sum(gt.astype(jnp.int32), axis=1, keepdims=True)
        eq = bits == t
        need = K - n_gt
        col = jax.lax.broadcasted_iota(jnp.int32, (BLK, S), 1)
        lo2 = jnp.zeros((BLK, 1), jnp.int32)
        hi2 = jnp.full((BLK, 1), S, jnp.int32)
        for _ in range(11):
            mid2 = lo2 + jax.lax.shift_right_logical(hi2 - lo2, 1)
            f = jnp.sum(jnp.logical_and(eq, col < mid2).astype(jnp.int32),
                        axis=1, keepdims=True)
            ge2 = f >= need
            hi2 = jnp.where(ge2, mid2, hi2)
            lo2 = jnp.where(ge2, lo2, mid2)
        tie_sel = eq & (col < hi2)
        topk = gt | tie_sel
        t_pos = t > 0
        pos_case = jnp.logical_and(jnp.logical_not(t_pos), n_gt > 0)
        zero_case = jnp.logical_and(jnp.logical_not(t_pos), n_gt == 0)
        sel = (jnp.logical_and(t_pos, topk)
               | jnp.logical_and(pos_case, bits > 0)
               | jnp.logical_and(zero_case, tie_sel))
        cnt_valid = jnp.where(pos_case, n_gt, K).astype(jnp.float32)
        row = (row_base
               + jax.lax.broadcasted_iota(jnp.int32, (BLK, 1), 0))
        row_valid = (row < len_ref[0, 0]).astype(jnp.float32)
        r = row_valid / cnt_valid
        contrib = jnp.sum(sel.astype(jnp.float32) * r, axis=0,
                          keepdims=True)
        c_ref[...] += contrib

    @pl.when(jnp.logical_and(h == H // 8 - 1, b >= 1))
    def _():
        bits = jax.lax.bitcast_convert_type(prev_ref[...], jnp.int32)
        _finalize(bits, lo_ref[...], (b - 1) * BLK)

    @pl.when(jnp.logical_and(h == H // 8 - 1, b < NBLK))
    def _():
        prev_ref[...] = acc_ref[...]
        lo_ref[...] = jnp.zeros((BLK, 1), jnp.int32)
        hi_ref[...] = jnp.full((BLK, 1), HI_BITS, jnp.int32)


def _tail_body(hid_ref, c_ref, len_ref, eta_ref, tW_ref, tb_ref, fW_ref,
               fb_ref, g_ref, bln_ref, oW_ref, ob_ref, y_ref):
    eta = eta_ref[0, 0]
    len_i = len_ref[0, 0]
    len_f = jnp.maximum(len_i, 1).astype(jnp.float32)
    col = jax.lax.broadcasted_iota(jnp.int32, (1, S), 1)
    valid = (col < len_i).astype(jnp.float32)
    u = (eta * valid + (1.0 - eta) * c_ref[...]) / len_f
    su = jnp.sum(u)
    s_vec = jnp.dot(u, hid_ref[0, :, :],
                    preferred_element_type=jnp.float32)
    gm = jnp.dot(s_vec, tW_ref[...],
                 preferred_element_type=jnp.float32) + su * tb_ref[...]
    t1 = jnp.tanh(gm)
    o = jnp.dot(t1, fW_ref[...],
                preferred_element_type=jnp.float32) + fb_ref[...]
    m = jnp.mean(o, axis=-1, keepdims=True)
    v = jnp.mean((o - m) ** 2, axis=-1, keepdims=True)
    ln = (o - m) * jax.lax.rsqrt(v + LN_EPS) * g_ref[...] + bln_ref[...]
    y_ref[...] = jnp.tanh(
        jnp.dot(ln, oW_ref[...], preferred_element_type=jnp.float32)
        + ob_ref[...])


def _module(hidden, attention, len_2d, trans_W, trans_b, fc_W, fc_b, ln_g,
            ln_b, eta_2d, out_W, out_b):
    c = pl.pallas_call(
        _select_body,
        grid=(NBLK + 1, H // 8),
        in_specs=[
            pl.BlockSpec((1, 8, BLK, S),
                         lambda b, h: (0, h, jnp.minimum(b, NBLK - 1), 0)),
            pl.BlockSpec(memory_space=pltpu.SMEM),
        ],
        out_specs=pl.BlockSpec((1, S), lambda b, h: (0, 0)),
        out_shape=jax.ShapeDtypeStruct((1, S), jnp.float32),
        scratch_shapes=[pltpu.VMEM((BLK, S), jnp.float32),
                        pltpu.VMEM((BLK, S), jnp.float32),
                        pltpu.VMEM((BLK, 1), jnp.int32),
                        pltpu.VMEM((BLK, 1), jnp.int32)],
    )(attention, len_2d)

    y = pl.pallas_call(
        _tail_body,
        in_specs=[
            pl.BlockSpec((1, S, D), lambda: (0, 0, 0)),
            pl.BlockSpec((1, S), lambda: (0, 0)),
            pl.BlockSpec(memory_space=pltpu.SMEM),
            pl.BlockSpec(memory_space=pltpu.SMEM),
            pl.BlockSpec((D, P), lambda: (0, 0)),
            pl.BlockSpec((1, P), lambda: (0, 0)),
            pl.BlockSpec((P, P), lambda: (0, 0)),
            pl.BlockSpec((1, P), lambda: (0, 0)),
            pl.BlockSpec((1, P), lambda: (0, 0)),
            pl.BlockSpec((1, P), lambda: (0, 0)),
            pl.BlockSpec((P, P), lambda: (0, 0)),
            pl.BlockSpec((1, P), lambda: (0, 0)),
        ],
        out_specs=pl.BlockSpec((1, P), lambda: (0, 0)),
        out_shape=jax.ShapeDtypeStruct((1, P), jnp.float32),
    )(hidden, c, len_2d, eta_2d, trans_W, trans_b.reshape(1, P), fc_W,
      fc_b.reshape(1, P), ln_g.reshape(1, P), ln_b.reshape(1, P), out_W,
      out_b.reshape(1, P))
    return y


def kernel(word_embed, word_attention, semantic_embed, semantic_attention,
           lengths, w_trans_W, w_trans_b, w_fc_W, w_fc_b, w_ln_g, w_ln_b,
           w_eta, s_trans_W, s_trans_b, s_fc_W, s_fc_b, s_ln_g, s_ln_b,
           s_eta, word_fc_W, word_fc_b, sem_fc_W, sem_fc_b):
    len_2d = lengths.reshape(1, 1)
    word_out = _module(word_embed, word_attention, len_2d, w_trans_W,
                       w_trans_b, w_fc_W, w_fc_b, w_ln_g, w_ln_b,
                       w_eta.reshape(1, 1), word_fc_W, word_fc_b)
    sem_out = _module(semantic_embed, semantic_attention, len_2d, s_trans_W,
                      s_trans_b, s_fc_W, s_fc_b, s_ln_g, s_ln_b,
                      s_eta.reshape(1, 1), sem_fc_W, sem_fc_b)
    return (word_out, sem_out)
